# baseline (speedup 1.0000x reference)
import functools

import jax
import jax.numpy as jnp
import numpy as np
from jax import lax
from jax.experimental import pallas as pl
from jax.experimental.pallas import tpu as pltpu
from jax.experimental.pallas import tpu_sc as plsc

NC, NS, L = 2, 16, 16
NW = NC * NS
ROWS, COLS = 4096, 200
HID = 32
EMBED_NUM = 100000
HALF = EMBED_NUM // 2

TR = COLS // 8
TC_ = HID // 8
NBUF = 4

_mesh = plsc.VectorSubcoreMesh(core_axis_name="c", subcore_axis_name="s")


@functools.partial(
    pl.kernel,
    out_type=jax.ShapeDtypeStruct((TR, NW, 8, 128), jnp.int32),
    mesh=_mesh,
    scratch_types=[
        pltpu.VMEM((TR, 8, 128), jnp.float32),
        pltpu.VMEM((TR, 8, 128), jnp.int32),
        pltpu.SemaphoreType.DMA,
        pltpu.SemaphoreType.DMA,
    ],
)
def _quantize(ang_hbm, q_hbm, av, qv, isem, osem):
    wid = lax.axis_index("s") * NC + lax.axis_index("c")

    ins = [pltpu.async_copy(
        ang_hbm.at[pl.ds(t * 8, 8), pl.ds(wid * 128, 128)], av.at[t], isem)
        for t in range(TR)]
    for c in ins:
        c.wait()

    pi = jnp.float32(np.pi)

    def quant(t, carry):
        for ri in range(8):
            for g in range(8):
                x = av[t, ri, pl.ds(g * L, L)]
                v = (x / pi + 1.0) * jnp.float32(HALF)
                v = jnp.minimum(jnp.maximum(v, 0.0), jnp.float32(EMBED_NUM - 1))
                qv[t, ri, pl.ds(g * L, L)] = v.astype(jnp.int32)
        return carry

    lax.fori_loop(0, TR, quant, 0)

    outs = [pltpu.async_copy(qv.at[t], q_hbm.at[t, wid], osem)
            for t in range(TR)]
    for c in outs:
        c.wait()


@functools.partial(
    pl.kernel,
    out_type=jax.ShapeDtypeStruct((COLS, TC_, NW, 1024), jnp.float32),
    mesh=_mesh,
    scratch_types=[
        pltpu.VMEM((TR, 8, 128), jnp.int32),
        pltpu.VMEM((NBUF, 128, HID), jnp.float32),
        [pltpu.VMEM((HID * 128,), jnp.float32)] * NBUF,
        pltpu.SemaphoreType.DMA,
        [pltpu.SemaphoreType.DMA] * NBUF,
        [pltpu.SemaphoreType.DMA] * NBUF,
    ],
    compiler_params=pltpu.CompilerParams(use_tc_tiling_on_sc=False,
                                         needs_layout_passes=False,
                                         disable_bounds_checks=True),
)
def _embed(q_hbm, table_hbm, out_hbm, idx_v, rows_v, trans_bufs,
           asem, gsems, wsems):
    wid = lax.axis_index("s") * NC + lax.axis_index("c")

    ins = [pltpu.async_copy(q_hbm.at[t, wid], idx_v.at[t], asem)
           for t in range(TR)]
    for c in ins:
        c.wait()

    iota = lax.iota(jnp.int32, L)
    row_ids = [iota + g * L for g in range(8)]

    def g_copy(j, b):
        return pltpu.make_async_copy(
            table_hbm.at[idx_v.at[j // 8, j % 8]],
            rows_v.at[b],
            gsems[b],
        )

    def w_copies(j, b):
        return [pltpu.make_async_copy(trans_bufs[b].at[pl.ds(tc * 1024, 1024)],
                                      out_hbm.at[j, tc, wid],
                                      wsems[b])
                for tc in range(TC_)]

    for b in range(NBUF):
        g_copy(b, b).start()

    def body(step, carry):
        j0 = step * NBUF
        for b in range(NBUF):
            j = j0 + b
            g_copy(j, b).wait()

            @pl.when(step > 0)
            def _():
                for c in w_copies(j, b):
                    c.wait()

            rows = rows_v.at[b]
            trans = trans_bufs[b]

            @plsc.parallel_loop(0, HID, step=1, unroll=32)
            def _(c):
                cv = lax.broadcast_in_dim(c, (L,), ())
                dcol = (cv + iota) & (HID - 1)
                dbase = dcol * 128
                for g in range(8):
                    v = plsc.load_gather(rows, [row_ids[g], dcol])
                    plsc.store_scatter(trans, [dbase + row_ids[g]], v)

            for c in w_copies(j, b):
                c.start()

            @pl.when(j + NBUF < COLS)
            def _():
                g_copy(j + NBUF, b).start()

        return carry

    lax.fori_loop(0, COLS // NBUF, body, 0)

    for b in range(NBUF):
        for c in w_copies(0, b):
            c.wait()


def kernel(index, weight):
    q4 = _quantize(index.T)
    out = _embed(q4, weight)
    out5 = out.reshape(COLS, TC_, NW, 8, 128)
    return out5.transpose(2, 4, 0, 1, 3).reshape(ROWS, COLS, HID)


# speedup vs baseline: 2.4058x; 2.4058x over previous
import functools

import jax
import jax.numpy as jnp
import numpy as np
from jax import lax
from jax.experimental import pallas as pl
from jax.experimental.pallas import tpu as pltpu
from jax.experimental.pallas import tpu_sc as plsc

NC, NS, L = 2, 16, 16
NW = NC * NS
ROWS, COLS = 4096, 200
HID = 32
EMBED_NUM = 100000
HALF = EMBED_NUM // 2

TR = COLS // 8
TC_ = HID // 8
NBUF = 4

_mesh = plsc.VectorSubcoreMesh(core_axis_name="c", subcore_axis_name="s")


@functools.partial(
    pl.kernel,
    out_type=jax.ShapeDtypeStruct((TR, NW, 8, 128), jnp.int32),
    mesh=_mesh,
    scratch_types=[
        pltpu.VMEM((TR, 8, 128), jnp.float32),
        pltpu.VMEM((TR, 8, 128), jnp.int32),
        pltpu.SemaphoreType.DMA,
        pltpu.SemaphoreType.DMA,
    ],
)
def _quantize(ang_hbm, q_hbm, av, qv, isem, osem):
    wid = lax.axis_index("s") * NC + lax.axis_index("c")

    ins = [pltpu.async_copy(
        ang_hbm.at[pl.ds(t * 8, 8), pl.ds(wid * 128, 128)], av.at[t], isem)
        for t in range(TR)]
    for c in ins:
        c.wait()

    pi = jnp.float32(np.pi)

    def quant(t, carry):
        for ri in range(8):
            for g in range(8):
                x = av[t, ri, pl.ds(g * L, L)]
                v = (x / pi + 1.0) * jnp.float32(HALF)
                v = jnp.minimum(jnp.maximum(v, 0.0), jnp.float32(EMBED_NUM - 1))
                qv[t, ri, pl.ds(g * L, L)] = v.astype(jnp.int32)
        return carry

    lax.fori_loop(0, TR, quant, 0)

    outs = [pltpu.async_copy(qv.at[t], q_hbm.at[t, wid], osem)
            for t in range(TR)]
    for c in outs:
        c.wait()


@functools.partial(
    pl.kernel,
    out_type=jax.ShapeDtypeStruct((COLS, TC_, NW, 1024), jnp.float32),
    mesh=_mesh,
    scratch_types=[
        pltpu.VMEM((TR, 8, 128), jnp.int32),
        pltpu.VMEM((NBUF, 128, HID), jnp.float32),
        [pltpu.VMEM((HID * 128,), jnp.float32)] * NBUF,
        pltpu.SemaphoreType.DMA,
        [pltpu.SemaphoreType.DMA] * NBUF,
        [pltpu.SemaphoreType.DMA] * NBUF,
    ],
    compiler_params=pltpu.CompilerParams(use_tc_tiling_on_sc=False,
                                         needs_layout_passes=False,
                                         disable_bounds_checks=True),
)
def _embed(q_hbm, table_hbm, out_hbm, idx_v, rows_v, trans_bufs,
           asem, gsems, wsems):
    wid = lax.axis_index("s") * NC + lax.axis_index("c")

    ins = [pltpu.async_copy(q_hbm.at[t, wid], idx_v.at[t], asem)
           for t in range(TR)]
    for c in ins:
        c.wait()

    iota = lax.iota(jnp.int32, L)
    row_ids = [iota + g * L for g in range(8)]

    def g_copy(j, b):
        return pltpu.make_async_copy(
            table_hbm.at[idx_v.at[j // 8, j % 8]],
            rows_v.at[b],
            gsems[b],
        )

    def w_copies(j, b):
        return [pltpu.make_async_copy(trans_bufs[b].at[pl.ds(tc * 1024, 1024)],
                                      out_hbm.at[j, tc, wid],
                                      wsems[b])
                for tc in range(TC_)]

    for b in range(NBUF):
        g_copy(b, b).start()

    def body(step, carry):
        j0 = step * NBUF
        for b in range(NBUF):
            j = j0 + b
            g_copy(j, b).wait()

            @pl.when(step > 0)
            def _():
                for c in w_copies(j, b):
                    c.wait()

            rows = rows_v.at[b]
            trans = trans_bufs[b]

            @plsc.parallel_loop(0, HID, step=1, unroll=16)
            def _(c):
                cv = lax.broadcast_in_dim(c, (L,), ())
                dcol = (cv + iota) & (HID - 1)
                dbase = dcol * 128
                for g in range(8):
                    v = plsc.load_gather(rows, [row_ids[g], dcol])
                    plsc.store_scatter(trans, [dbase + row_ids[g]], v)

            for c in w_copies(j, b):
                c.start()

            @pl.when(j + NBUF < COLS)
            def _():
                g_copy(j + NBUF, b).start()

        return carry

    lax.fori_loop(0, COLS // NBUF, body, 0)

    for b in range(NBUF):
        for c in w_copies(0, b):
            c.wait()


def kernel(index, weight):
    q4 = _quantize(index.T)
    out = _embed(q4, weight)
    out5 = out.reshape(COLS, TC_, NW, 8, 128)
    return out5.transpose(2, 4, 0, 1, 3).reshape(ROWS, COLS, HID)
